# baseline (speedup 1.0000x reference)
import jax
import jax.numpy as jnp
from jax import lax
from jax.experimental import pallas as pl
from jax.experimental.pallas import tpu as pltpu
from jax.experimental.pallas import tpu_sc as plsc

BATCH = 16384
D = 32
GROUP = 128
RPG = GROUP // D
HID = 64
NC, NS = 2, 16
NW = NC * NS
ROWS_PER_W = BATCH // NW
CHUNK = 128
NCHUNK = ROWS_PER_W // CHUNK
IDX_ROWS = BATCH // CHUNK


def _gather_body(uidx_hbm, midx_hbm, u_tab, m_tab, u_out, m_out,
                 uidx_v, midx_v, ubuf, mbuf, sem):
    wid = lax.axis_index("s") * NC + lax.axis_index("c")
    base = wid * NCHUNK
    pltpu.sync_copy(uidx_hbm.at[pl.ds(base, NCHUNK)], uidx_v)
    pltpu.sync_copy(midx_hbm.at[pl.ds(base, NCHUNK)], midx_v)
    g = []
    for j in range(NCHUNK):
        p = j % 2
        if j >= 2:
            g[j - 2][0].wait()
            g[j - 2][1].wait()
            pltpu.sync_copy(ubuf.at[p], u_out.at[base + j - 2])
            pltpu.sync_copy(mbuf.at[p], m_out.at[base + j - 2])
        g.append((pltpu.async_copy(u_tab.at[uidx_v.at[j]], ubuf.at[p], sem),
                  pltpu.async_copy(m_tab.at[midx_v.at[j]], mbuf.at[p], sem)))
    for j in (NCHUNK - 2, NCHUNK - 1):
        p = j % 2
        g[j][0].wait()
        g[j][1].wait()
        pltpu.sync_copy(ubuf.at[p], u_out.at[base + j])
        pltpu.sync_copy(mbuf.at[p], m_out.at[base + j])


def _sc_gather(uidx, midx, u_tab, m_tab):
    mesh = plsc.VectorSubcoreMesh(core_axis_name="c", subcore_axis_name="s",
                                  num_cores=NC, num_subcores=NS)
    out_t = (jax.ShapeDtypeStruct((IDX_ROWS, CHUNK, GROUP), jnp.float32),
             jax.ShapeDtypeStruct((IDX_ROWS, CHUNK, GROUP), jnp.float32))
    scratch = [
        pltpu.VMEM((NCHUNK, CHUNK), jnp.int32),
        pltpu.VMEM((NCHUNK, CHUNK), jnp.int32),
        pltpu.VMEM((2, CHUNK, GROUP), jnp.float32),
        pltpu.VMEM((2, CHUNK, GROUP), jnp.float32),
        pltpu.SemaphoreType.DMA,
    ]
    return pl.kernel(_gather_body, out_type=out_t, mesh=mesh,
                     scratch_types=scratch)(uidx, midx, u_tab, m_tab)


def _mlp_body(up_ref, mp_ref, su_ref, sm_ref, w1_ref, b1_ref, w2_ref, b2_ref,
              o_ref):
    su = su_ref[...]
    sm = sm_ref[...]
    u = up_ref[:, 0:D]
    m = mp_ref[:, 0:D]
    for s in range(1, RPG):
        u = jnp.where(su == s, up_ref[:, s * D:(s + 1) * D], u)
        m = jnp.where(sm == s, mp_ref[:, s * D:(s + 1) * D], m)
    h = jnp.dot(u, w1_ref[0:D, :], preferred_element_type=jnp.float32)
    h = h + jnp.dot(m, w1_ref[D:2 * D, :], preferred_element_type=jnp.float32)
    h = jnp.maximum(h + b1_ref[...], 0.0)
    o = jnp.dot(h, w2_ref[...], preferred_element_type=jnp.float32) + b2_ref[...]
    o_ref[...] = jax.nn.sigmoid(o) * 5.5


def _mlp(u_pad, m_pad, su, sm, w1, b1, w2, b2, block_rows=2048):
    grid = (BATCH // block_rows,)
    return pl.pallas_call(
        _mlp_body,
        grid=grid,
        in_specs=[
            pl.BlockSpec((block_rows, GROUP), lambda i: (i, 0)),
            pl.BlockSpec((block_rows, GROUP), lambda i: (i, 0)),
            pl.BlockSpec((block_rows, 1), lambda i: (i, 0)),
            pl.BlockSpec((block_rows, 1), lambda i: (i, 0)),
            pl.BlockSpec((2 * D, HID), lambda i: (0, 0)),
            pl.BlockSpec((1, HID), lambda i: (0, 0)),
            pl.BlockSpec((HID, 1), lambda i: (0, 0)),
            pl.BlockSpec((1, 1), lambda i: (0, 0)),
        ],
        out_specs=pl.BlockSpec((block_rows, 1), lambda i: (i, 0)),
        out_shape=jax.ShapeDtypeStruct((BATCH, 1), jnp.float32),
    )(u_pad, m_pad, su, sm, w1, b1.reshape(1, HID), w2, b2.reshape(1, 1))


def kernel(cats, u_table, m_table, w1, b1, w2, b2):
    cats = cats.astype(jnp.int32)
    users = cats[:, 0]
    movies = cats[:, 1]
    uidx = (users // RPG).reshape(IDX_ROWS, CHUNK)
    midx = (movies // RPG).reshape(IDX_ROWS, CHUNK)
    su = (users % RPG).reshape(BATCH, 1)
    sm = (movies % RPG).reshape(BATCH, 1)
    u_tab = u_table.reshape(u_table.shape[0] // RPG, GROUP)
    m_tab = m_table.reshape(m_table.shape[0] // RPG, GROUP)
    u_pad, m_pad = _sc_gather(uidx, midx, u_tab, m_tab)
    u_pad = u_pad.reshape(BATCH, GROUP)
    m_pad = m_pad.reshape(BATCH, GROUP)
    return _mlp(u_pad, m_pad, su, sm, w1, b1, w2, b2)


# speedup vs baseline: 1.5501x; 1.5501x over previous
import jax
import jax.numpy as jnp
from jax import lax
from jax.experimental import pallas as pl
from jax.experimental.pallas import tpu as pltpu
from jax.experimental.pallas import tpu_sc as plsc

BATCH = 16384
D = 32
HID = 64
NC, NS = 2, 16
NW = NC * NS
ROWS_PER_W = BATCH // NW
CHUNK = 128
NCHUNK = ROWS_PER_W // CHUNK
IDX_ROWS = BATCH // CHUNK
LANES = 16
NGROUP = CHUNK // LANES


def _gather_body(uidx_hbm, midx_hbm, u_tab, m_tab, u_out, m_out,
                 uidx_v, midx_v, ubuf, mbuf, sem):
    wid = lax.axis_index("s") * NC + lax.axis_index("c")
    base = wid * ROWS_PER_W
    pltpu.sync_copy(uidx_hbm.at[pl.ds(base, ROWS_PER_W)], uidx_v)
    pltpu.sync_copy(midx_hbm.at[pl.ds(base, ROWS_PER_W)], midx_v)
    lane = lax.iota(jnp.int32, LANES)

    def fire(j, p):
        def group(g, carry):
            off = pl.multiple_of(j * CHUNK + g * LANES, LANES)
            vu = uidx_v[pl.ds(off, LANES)]
            vm = midx_v[pl.ds(off, LANES)]
            for l in range(LANES):
                ru = jnp.sum(jnp.where(lane == l, vu, 0))
                rm = jnp.sum(jnp.where(lane == l, vm, 0))
                row = g * LANES + l
                pltpu.make_async_copy(u_tab.at[pl.ds(ru, 1)],
                                      ubuf.at[p, pl.ds(row, 1)], sem).start()
                pltpu.make_async_copy(m_tab.at[pl.ds(rm, 1)],
                                      mbuf.at[p, pl.ds(row, 1)], sem).start()
            return carry
        lax.fori_loop(0, NGROUP, group, 0)

    def drain_and_copy_out(j, p):
        out_row = wid * NCHUNK + j
        pltpu.make_async_copy(u_out.at[out_row], ubuf.at[p], sem).wait()
        pltpu.make_async_copy(m_out.at[out_row], mbuf.at[p], sem).wait()
        pltpu.sync_copy(ubuf.at[p], u_out.at[out_row])
        pltpu.sync_copy(mbuf.at[p], m_out.at[out_row])

    fire(0, 0)
    for j in range(1, NCHUNK):
        fire(j, j % 2)
        drain_and_copy_out(j - 1, (j - 1) % 2)
    drain_and_copy_out(NCHUNK - 1, (NCHUNK - 1) % 2)


def _sc_gather(uidx, midx, u_tab, m_tab):
    mesh = plsc.VectorSubcoreMesh(core_axis_name="c", subcore_axis_name="s",
                                  num_cores=NC, num_subcores=NS)
    out_t = (jax.ShapeDtypeStruct((IDX_ROWS, CHUNK, D), jnp.float32),
             jax.ShapeDtypeStruct((IDX_ROWS, CHUNK, D), jnp.float32))
    scratch = [
        pltpu.VMEM((ROWS_PER_W,), jnp.int32),
        pltpu.VMEM((ROWS_PER_W,), jnp.int32),
        pltpu.VMEM((2, CHUNK, D), jnp.float32),
        pltpu.VMEM((2, CHUNK, D), jnp.float32),
        pltpu.SemaphoreType.DMA,
    ]
    params = pltpu.CompilerParams(needs_layout_passes=False)
    return pl.kernel(_gather_body, out_type=out_t, mesh=mesh,
                     scratch_types=scratch,
                     compiler_params=params)(uidx, midx, u_tab, m_tab)


def _mlp_body(u_ref, m_ref, w1_ref, b1_ref, w2_ref, b2_ref, o_ref):
    h = jnp.dot(u_ref[...], w1_ref[0:D, :], preferred_element_type=jnp.float32)
    h = h + jnp.dot(m_ref[...], w1_ref[D:2 * D, :],
                    preferred_element_type=jnp.float32)
    h = jnp.maximum(h + b1_ref[...], 0.0)
    o = jnp.dot(h, w2_ref[...], preferred_element_type=jnp.float32) + b2_ref[...]
    o_ref[...] = jax.nn.sigmoid(o) * 5.5


def _mlp(u_emb, m_emb, w1, b1, w2, b2, block_rows=2048):
    grid = (BATCH // block_rows,)
    return pl.pallas_call(
        _mlp_body,
        grid=grid,
        in_specs=[
            pl.BlockSpec((block_rows, D), lambda i: (i, 0)),
            pl.BlockSpec((block_rows, D), lambda i: (i, 0)),
            pl.BlockSpec((2 * D, HID), lambda i: (0, 0)),
            pl.BlockSpec((1, HID), lambda i: (0, 0)),
            pl.BlockSpec((HID, 1), lambda i: (0, 0)),
            pl.BlockSpec((1, 1), lambda i: (0, 0)),
        ],
        out_specs=pl.BlockSpec((block_rows, 1), lambda i: (i, 0)),
        out_shape=jax.ShapeDtypeStruct((BATCH, 1), jnp.float32),
    )(u_emb, m_emb, w1, b1.reshape(1, HID), w2, b2.reshape(1, 1))


def kernel(cats, u_table, m_table, w1, b1, w2, b2):
    cats = cats.astype(jnp.int32)
    uidx = cats[:, 0]
    midx = cats[:, 1]
    u_emb, m_emb = _sc_gather(uidx, midx, u_table, m_table)
    u_emb = u_emb.reshape(BATCH, D)
    m_emb = m_emb.reshape(BATCH, D)
    return _mlp(u_emb, m_emb, w1, b1, w2, b2)


# speedup vs baseline: 1.5533x; 1.0021x over previous
import jax
import jax.numpy as jnp
from jax import lax
from jax.experimental import pallas as pl
from jax.experimental.pallas import tpu as pltpu
from jax.experimental.pallas import tpu_sc as plsc

BATCH = 16384
D = 32
HID = 64
NC, NS = 2, 16
NW = NC * NS
ROWS_PER_W = BATCH // NW
CHUNK = 128
NCHUNK = ROWS_PER_W // CHUNK
IDX_ROWS = BATCH // CHUNK
LANES = 16
NGROUP = CHUNK // LANES


def _gather_body(uidx_hbm, midx_hbm, u_tab, m_tab, u_out, m_out,
                 uidx_v, midx_v, ubuf, mbuf, sem):
    wid = lax.axis_index("s") * NC + lax.axis_index("c")
    base = wid * ROWS_PER_W
    pltpu.sync_copy(uidx_hbm.at[pl.ds(base, ROWS_PER_W)], uidx_v)
    pltpu.sync_copy(midx_hbm.at[pl.ds(base, ROWS_PER_W)], midx_v)
    lane = lax.iota(jnp.int32, LANES)

    def fire(j, p):
        def group(g, carry):
            off = pl.multiple_of(j * CHUNK + g * LANES, LANES)
            vu = uidx_v[pl.ds(off, LANES)]
            vm = midx_v[pl.ds(off, LANES)]
            for l in range(LANES):
                ru = vu[l]
                rm = vm[l]
                row = g * LANES + l
                pltpu.make_async_copy(u_tab.at[pl.ds(ru, 1)],
                                      ubuf.at[p, pl.ds(row, 1)], sem).start()
                pltpu.make_async_copy(m_tab.at[pl.ds(rm, 1)],
                                      mbuf.at[p, pl.ds(row, 1)], sem).start()
            return carry
        lax.fori_loop(0, NGROUP, group, 0)

    def drain_and_copy_out(j, p):
        out_row = wid * NCHUNK + j
        pltpu.make_async_copy(u_out.at[out_row], ubuf.at[p], sem).wait()
        pltpu.make_async_copy(m_out.at[out_row], mbuf.at[p], sem).wait()
        pltpu.sync_copy(ubuf.at[p], u_out.at[out_row])
        pltpu.sync_copy(mbuf.at[p], m_out.at[out_row])

    fire(0, 0)
    for j in range(1, NCHUNK):
        fire(j, j % 2)
        drain_and_copy_out(j - 1, (j - 1) % 2)
    drain_and_copy_out(NCHUNK - 1, (NCHUNK - 1) % 2)


def _sc_gather(uidx, midx, u_tab, m_tab):
    mesh = plsc.VectorSubcoreMesh(core_axis_name="c", subcore_axis_name="s",
                                  num_cores=NC, num_subcores=NS)
    out_t = (jax.ShapeDtypeStruct((IDX_ROWS, CHUNK, D), jnp.float32),
             jax.ShapeDtypeStruct((IDX_ROWS, CHUNK, D), jnp.float32))
    scratch = [
        pltpu.VMEM((ROWS_PER_W,), jnp.int32),
        pltpu.VMEM((ROWS_PER_W,), jnp.int32),
        pltpu.VMEM((2, CHUNK, D), jnp.float32),
        pltpu.VMEM((2, CHUNK, D), jnp.float32),
        pltpu.SemaphoreType.DMA,
    ]
    return pl.kernel(_gather_body, out_type=out_t, mesh=mesh,
                     scratch_types=scratch)(uidx, midx, u_tab, m_tab)


def _mlp_body(u_ref, m_ref, w1_ref, b1_ref, w2_ref, b2_ref, o_ref):
    h = jnp.dot(u_ref[...], w1_ref[0:D, :], preferred_element_type=jnp.float32)
    h = h + jnp.dot(m_ref[...], w1_ref[D:2 * D, :],
                    preferred_element_type=jnp.float32)
    h = jnp.maximum(h + b1_ref[...], 0.0)
    o = jnp.dot(h, w2_ref[...], preferred_element_type=jnp.float32) + b2_ref[...]
    o_ref[...] = jax.nn.sigmoid(o) * 5.5


def _mlp(u_emb, m_emb, w1, b1, w2, b2, block_rows=2048):
    grid = (BATCH // block_rows,)
    return pl.pallas_call(
        _mlp_body,
        grid=grid,
        in_specs=[
            pl.BlockSpec((block_rows, D), lambda i: (i, 0)),
            pl.BlockSpec((block_rows, D), lambda i: (i, 0)),
            pl.BlockSpec((2 * D, HID), lambda i: (0, 0)),
            pl.BlockSpec((1, HID), lambda i: (0, 0)),
            pl.BlockSpec((HID, 1), lambda i: (0, 0)),
            pl.BlockSpec((1, 1), lambda i: (0, 0)),
        ],
        out_specs=pl.BlockSpec((block_rows, 1), lambda i: (i, 0)),
        out_shape=jax.ShapeDtypeStruct((BATCH, 1), jnp.float32),
    )(u_emb, m_emb, w1, b1.reshape(1, HID), w2, b2.reshape(1, 1))


def kernel(cats, u_table, m_table, w1, b1, w2, b2):
    cats = cats.astype(jnp.int32)
    uidx = cats[:, 0]
    midx = cats[:, 1]
    u_emb, m_emb = _sc_gather(uidx, midx, u_table, m_table)
    u_emb = u_emb.reshape(BATCH, D)
    m_emb = m_emb.reshape(BATCH, D)
    return _mlp(u_emb, m_emb, w1, b1, w2, b2)
